# baseline (speedup 1.0000x reference)
import functools

import jax
import jax.numpy as jnp
from jax import lax
from jax.experimental import pallas as pl
from jax.experimental.pallas import tpu as pltpu
from jax.experimental.pallas import tpu_sc as plsc

E = 320000
H = 128
R = 500
NC = 2
NS = 16
NW = NC * NS
EPW = E // NW
K = 16
NCHUNK = EPW // K
G = K // 16
HW = H // 2

_EU = 4
_ILV = plsc.PackFormat.INTERLEAVED

_mesh = plsc.VectorSubcoreMesh(core_axis_name="c", subcore_axis_name="s")


@functools.partial(
    pl.kernel,
    mesh=_mesh,
    out_type=jax.ShapeDtypeStruct((E,), jnp.float32),
    compiler_params=pltpu.CompilerParams(needs_layout_passes=False,
                                         use_tc_tiling_on_sc=False),
    scratch_types=[
        pltpu.VMEM((EPW,), jnp.int32),
        pltpu.VMEM((EPW,), jnp.int32),
        pltpu.VMEM((EPW,), jnp.int32),
        pltpu.VMEM((EPW,), jnp.float32),
        pltpu.VMEM((R, HW), jnp.int32),
        pltpu.VMEM((K, HW), jnp.int32),
        pltpu.VMEM((K, HW), jnp.int32),
        pltpu.VMEM((K, HW), jnp.int32),
        pltpu.VMEM((K, HW), jnp.int32),
        pltpu.SemaphoreType.DMA,
        pltpu.SemaphoreType.DMA,
        pltpu.SemaphoreType.DMA,
        pltpu.SemaphoreType.DMA,
    ],
)
def _distmult_sc(src_hbm, dst_hbm, typ_hbm, z_hbm, rel_hbm, out_hbm,
                 sidx_v, didx_v, tidx_v, out_v, rl_all,
                 zsA, zdA, zsB, zdB,
                 ssA, sdA, ssB, sdB):
    wid = lax.axis_index("s") * NC + lax.axis_index("c")
    row16 = lax.iota(jnp.int32, 16)
    bufs = ((zsA, zdA, ssA, sdA), (zsB, zdB, ssB, sdB))

    base = wid * EPW
    pltpu.sync_copy(src_hbm.at[pl.ds(base, EPW)], sidx_v)
    pltpu.sync_copy(dst_hbm.at[pl.ds(base, EPW)], didx_v)
    pltpu.sync_copy(typ_hbm.at[pl.ds(base, EPW)], tidx_v)
    pltpu.sync_copy(rel_hbm, rl_all)

    def start(c, buf):
        zs, zd, s_s, s_d = buf
        off = c * K
        pltpu.async_copy(z_hbm.at[sidx_v.at[pl.ds(off, K)]], zs, s_s)
        pltpu.async_copy(z_hbm.at[didx_v.at[pl.ds(off, K)]], zd, s_d)

    def wait(c, buf):
        zs, zd, s_s, s_d = buf
        off = c * K
        pltpu.make_async_copy(z_hbm.at[sidx_v.at[pl.ds(off, K)]], zs, s_s).wait()
        pltpu.make_async_copy(z_hbm.at[didx_v.at[pl.ds(off, K)]], zd, s_d).wait()

    def compute(c, buf):
        zs, zd = buf[0], buf[1]

        def group_body(g, carry):
            gbase = g * 16

            def edge_blk(eb, acc_out):
                tvec = tidx_v[pl.ds(c * K + gbase + eb * _EU, 16)]
                for u in range(_EU):
                    e16 = eb * _EU + u
                    e = gbase + e16
                    tid = tvec[u]
                    prods = []
                    for t in range(H // 32):
                        sl = pl.ds(t * 16, 16)
                        s0, s1 = plsc.unpack(
                            plsc.bitcast(zs[e, sl], jnp.bfloat16), format=_ILV)
                        r0, r1 = plsc.unpack(
                            plsc.bitcast(rl_all[tid, sl], jnp.bfloat16),
                            format=_ILV)
                        d0, d1 = plsc.unpack(
                            plsc.bitcast(zd[e, sl], jnp.bfloat16), format=_ILV)
                        prods.append(s0 * r0 * d0)
                        prods.append(s1 * r1 * d1)
                    while len(prods) > 1:
                        prods = [prods[k] + prods[k + 1]
                                 for k in range(0, len(prods), 2)]
                    s = jnp.sum(prods[0])
                    acc_out = jnp.where(row16 == e16, s, acc_out)
                return acc_out

            acc_out = lax.fori_loop(0, 16 // _EU, edge_blk,
                                    jnp.zeros((16,), jnp.float32))
            out_v[pl.ds(c * K + gbase, 16)] = acc_out
            return carry

        lax.fori_loop(0, G, group_body, 0)

    start(0, bufs[0])

    def chunk_pair(i, carry):
        for par in range(2):
            c = 2 * i + par
            start(c + 1, bufs[(par + 1) % 2])
            wait(c, bufs[par])
            compute(c, bufs[par])
        return carry

    lax.fori_loop(0, (NCHUNK - 1) // 2, chunk_pair, 0)
    cl = NCHUNK - 1
    wait(cl, bufs[0])
    compute(cl, bufs[0])

    pltpu.sync_copy(out_v, out_hbm.at[pl.ds(base, EPW)])


def kernel(z, edge_index, edge_type, rel_emb):
    src = edge_index[0].astype(jnp.int32)
    dst = edge_index[1].astype(jnp.int32)
    typ = edge_type.astype(jnp.int32)
    zb = jax.lax.bitcast_convert_type(
        z.astype(jnp.bfloat16).reshape(z.shape[0], HW, 2), jnp.int32)
    rb = jax.lax.bitcast_convert_type(
        rel_emb.astype(jnp.bfloat16).reshape(rel_emb.shape[0], HW, 2),
        jnp.int32)
    return _distmult_sc(src, dst, typ, zb, rb)


# speedup vs baseline: 1.6534x; 1.6534x over previous
import functools

import jax
import jax.numpy as jnp
from jax import lax
from jax.experimental import pallas as pl
from jax.experimental.pallas import tpu as pltpu
from jax.experimental.pallas import tpu_sc as plsc

E = 320000
H = 128
R = 500
NN = 10000
NC = 2
NS = 16
NW = NC * NS
EPW = E // NW
K = 80
NCHUNK = EPW // K
G = K // 16
HW = H // 2

_EU = 4
_ILV = plsc.PackFormat.INTERLEAVED

_mesh = plsc.VectorSubcoreMesh(core_axis_name="c", subcore_axis_name="s")


@functools.partial(
    pl.kernel,
    mesh=_mesh,
    out_type=jax.ShapeDtypeStruct((E,), jnp.float32),
    compiler_params=pltpu.CompilerParams(needs_layout_passes=False,
                                         use_tc_tiling_on_sc=False),
    scratch_types=[
        pltpu.VMEM((EPW,), jnp.int32),
        pltpu.VMEM((EPW,), jnp.int32),
        pltpu.VMEM((EPW + 32,), jnp.int16),
        pltpu.VMEM((EPW,), jnp.float32),
        pltpu.VMEM((R, HW), jnp.int32),
        pltpu.VMEM_SHARED((NN, HW), jnp.int32),
        pltpu.VMEM((K, HW), jnp.int32),
        pltpu.VMEM((K, HW), jnp.int32),
        pltpu.VMEM((K, HW), jnp.int32),
        pltpu.VMEM((K, HW), jnp.int32),
        pltpu.SemaphoreType.DMA,
        pltpu.SemaphoreType.DMA,
        pltpu.SemaphoreType.DMA,
        pltpu.SemaphoreType.DMA,
    ],
)
def _distmult_sc(src_hbm, dst_hbm, typ_hbm, z_hbm, rel_hbm, out_hbm,
                 sidx_v, didx_v, tidx_v, out_v, rl_all, z_sh,
                 zsA, zdA, zsB, zdB,
                 ssA, sdA, ssB, sdB):
    wid = lax.axis_index("s") * NC + lax.axis_index("c")
    sid = lax.axis_index("s")
    row16 = lax.iota(jnp.int32, 16)
    bufs = ((zsA, zdA, ssA, sdA), (zsB, zdB, ssB, sdB))

    base = wid * EPW
    pltpu.sync_copy(src_hbm.at[pl.ds(base, EPW)], sidx_v)
    pltpu.sync_copy(dst_hbm.at[pl.ds(base, EPW)], didx_v)
    pltpu.sync_copy(typ_hbm.at[pl.ds(base, EPW)], tidx_v.at[pl.ds(0, EPW)])
    pltpu.sync_copy(rel_hbm, rl_all)
    zstride = NN // NS
    pltpu.sync_copy(z_hbm.at[pl.ds(sid * zstride, zstride)],
                    z_sh.at[pl.ds(sid * zstride, zstride)])
    plsc.subcore_barrier()

    def start(c, buf):
        zs, zd, s_s, s_d = buf
        off = c * K
        pltpu.async_copy(z_sh.at[sidx_v.at[pl.ds(off, K)]], zs, s_s)
        pltpu.async_copy(z_sh.at[didx_v.at[pl.ds(off, K)]], zd, s_d)

    def wait(c, buf):
        zs, zd, s_s, s_d = buf
        off = c * K
        pltpu.make_async_copy(z_sh.at[sidx_v.at[pl.ds(off, K)]], zs, s_s).wait()
        pltpu.make_async_copy(z_sh.at[didx_v.at[pl.ds(off, K)]], zd, s_d).wait()

    def compute(c, buf):
        zs, zd = buf[0], buf[1]

        def group_body(g, carry):
            gbase = g * 16

            def edge_blk(eb, acc_out):
                tvec = tidx_v[pl.ds(c * K + gbase + eb * _EU, 32)]
                tv0, tv1 = plsc.unpack(tvec, format=_ILV)
                for u in range(_EU):
                    e16 = eb * _EU + u
                    e = gbase + e16
                    tid = (tv0, tv1)[u % 2][u // 2]
                    prods = []
                    for t in range(H // 32):
                        sl = pl.ds(t * 16, 16)
                        s0, s1 = plsc.unpack(
                            plsc.bitcast(zs[e, sl], jnp.bfloat16), format=_ILV)
                        r0, r1 = plsc.unpack(
                            plsc.bitcast(rl_all[tid, sl], jnp.bfloat16),
                            format=_ILV)
                        d0, d1 = plsc.unpack(
                            plsc.bitcast(zd[e, sl], jnp.bfloat16), format=_ILV)
                        prods.append(s0 * r0 * d0)
                        prods.append(s1 * r1 * d1)
                    while len(prods) > 1:
                        prods = [prods[k] + prods[k + 1]
                                 for k in range(0, len(prods), 2)]
                    s = jnp.sum(prods[0])
                    acc_out = jnp.where(row16 == e16, s, acc_out)
                return acc_out

            acc_out = lax.fori_loop(0, 16 // _EU, edge_blk,
                                    jnp.zeros((16,), jnp.float32))
            out_v[pl.ds(c * K + gbase, 16)] = acc_out
            return carry

        lax.fori_loop(0, G, group_body, 0)

    start(0, bufs[0])

    def chunk_pair(i, carry):
        for par in range(2):
            c = 2 * i + par
            start(c + 1, bufs[(par + 1) % 2])
            wait(c, bufs[par])
            compute(c, bufs[par])
        return carry

    lax.fori_loop(0, (NCHUNK - 1) // 2, chunk_pair, 0)
    cl = NCHUNK - 1
    wait(cl, bufs[0])
    compute(cl, bufs[0])

    pltpu.sync_copy(out_v, out_hbm.at[pl.ds(base, EPW)])


def kernel(z, edge_index, edge_type, rel_emb):
    src = edge_index[0].astype(jnp.int32)
    dst = edge_index[1].astype(jnp.int32)
    typ = edge_type.astype(jnp.int16)
    zb = jax.lax.bitcast_convert_type(
        z.astype(jnp.bfloat16).reshape(z.shape[0], HW, 2), jnp.int32)
    rb = jax.lax.bitcast_convert_type(
        rel_emb.astype(jnp.bfloat16).reshape(rel_emb.shape[0], HW, 2),
        jnp.int32)
    return _distmult_sc(src, dst, typ, zb, rb)


# speedup vs baseline: 1.6555x; 1.0013x over previous
import functools

import jax
import jax.numpy as jnp
from jax import lax
from jax.experimental import pallas as pl
from jax.experimental.pallas import tpu as pltpu
from jax.experimental.pallas import tpu_sc as plsc

E = 320000
H = 128
R = 500
NN = 10000
NC = 2
NS = 16
NW = NC * NS
EPW = E // NW
K = 80
NCHUNK = EPW // K
G = K // 16
HW = H // 2

_EU = 4
_ILV = plsc.PackFormat.INTERLEAVED

_mesh = plsc.VectorSubcoreMesh(core_axis_name="c", subcore_axis_name="s")


@functools.partial(
    pl.kernel,
    mesh=_mesh,
    out_type=jax.ShapeDtypeStruct((E,), jnp.float32),
    compiler_params=pltpu.CompilerParams(needs_layout_passes=False,
                                         use_tc_tiling_on_sc=False),
    scratch_types=[
        pltpu.VMEM((EPW,), jnp.int32),
        pltpu.VMEM((EPW,), jnp.int32),
        pltpu.VMEM((EPW + 32,), jnp.int16),
        pltpu.VMEM((EPW,), jnp.float32),
        pltpu.VMEM((R, HW), jnp.int32),
        pltpu.VMEM((K, HW), jnp.int32),
        pltpu.VMEM((K, HW), jnp.int32),
        pltpu.VMEM((K, HW), jnp.int32),
        pltpu.VMEM((K, HW), jnp.int32),
        pltpu.VMEM((K, HW), jnp.int32),
        pltpu.VMEM((K, HW), jnp.int32),
        pltpu.SemaphoreType.DMA,
        pltpu.SemaphoreType.DMA,
        pltpu.SemaphoreType.DMA,
        pltpu.SemaphoreType.DMA,
        pltpu.SemaphoreType.DMA,
        pltpu.SemaphoreType.DMA,
    ],
)
def _distmult_sc(src_hbm, dst_hbm, typ_hbm, z_hbm, rel_hbm, out_hbm,
                 sidx_v, didx_v, tidx_v, out_v, rl_all,
                 zsA, zdA, zsB, zdB, zsC, zdC,
                 ssA, sdA, ssB, sdB, ssC, sdC):
    wid = lax.axis_index("s") * NC + lax.axis_index("c")
    row16 = lax.iota(jnp.int32, 16)
    bufs = ((zsA, zdA, ssA, sdA), (zsB, zdB, ssB, sdB), (zsC, zdC, ssC, sdC))

    base = wid * EPW
    pltpu.sync_copy(src_hbm.at[pl.ds(base, EPW)], sidx_v)
    pltpu.sync_copy(dst_hbm.at[pl.ds(base, EPW)], didx_v)
    pltpu.sync_copy(typ_hbm.at[pl.ds(base, EPW)], tidx_v.at[pl.ds(0, EPW)])
    pltpu.sync_copy(rel_hbm, rl_all)

    def start(c, buf):
        zs, zd, s_s, s_d = buf
        off = c * K
        pltpu.async_copy(z_hbm.at[sidx_v.at[pl.ds(off, K)]], zs, s_s)
        pltpu.async_copy(z_hbm.at[didx_v.at[pl.ds(off, K)]], zd, s_d)

    def wait(c, buf):
        zs, zd, s_s, s_d = buf
        off = c * K
        pltpu.make_async_copy(z_hbm.at[sidx_v.at[pl.ds(off, K)]], zs, s_s).wait()
        pltpu.make_async_copy(z_hbm.at[didx_v.at[pl.ds(off, K)]], zd, s_d).wait()

    def compute(c, buf):
        zs, zd = buf[0], buf[1]

        def group_body(g, carry):
            gbase = g * 16

            def edge_blk(eb, acc_out):
                tvec = tidx_v[pl.ds(c * K + gbase + eb * _EU, 32)]
                tv0, tv1 = plsc.unpack(tvec, format=_ILV)
                for u in range(_EU):
                    e16 = eb * _EU + u
                    e = gbase + e16
                    tid = (tv0, tv1)[u % 2][u // 2]
                    prods = []
                    for t in range(H // 32):
                        sl = pl.ds(t * 16, 16)
                        s0, s1 = plsc.unpack(
                            plsc.bitcast(zs[e, sl], jnp.bfloat16), format=_ILV)
                        r0, r1 = plsc.unpack(
                            plsc.bitcast(rl_all[tid, sl], jnp.bfloat16),
                            format=_ILV)
                        d0, d1 = plsc.unpack(
                            plsc.bitcast(zd[e, sl], jnp.bfloat16), format=_ILV)
                        prods.append(s0 * r0 * d0)
                        prods.append(s1 * r1 * d1)
                    while len(prods) > 1:
                        prods = [prods[k] + prods[k + 1]
                                 for k in range(0, len(prods), 2)]
                    s = jnp.sum(prods[0])
                    acc_out = jnp.where(row16 == e16, s, acc_out)
                return acc_out

            acc_out = lax.fori_loop(0, 16 // _EU, edge_blk,
                                    jnp.zeros((16,), jnp.float32))
            out_v[pl.ds(c * K + gbase, 16)] = acc_out
            return carry

        lax.fori_loop(0, G, group_body, 0)

    start(0, bufs[0])
    start(1, bufs[1])

    def chunk_trip(i, carry):
        for par in range(3):
            c = 3 * i + par
            start(c + 2, bufs[(par + 2) % 3])
            wait(c, bufs[par])
            compute(c, bufs[par])
        return carry

    lax.fori_loop(0, (NCHUNK - 2) // 3, chunk_trip, 0)
    for cl in (NCHUNK - 2, NCHUNK - 1):
        wait(cl, bufs[cl % 3])
        compute(cl, bufs[cl % 3])

    pltpu.sync_copy(out_v, out_hbm.at[pl.ds(base, EPW)])


def kernel(z, edge_index, edge_type, rel_emb):
    src = edge_index[0].astype(jnp.int32)
    dst = edge_index[1].astype(jnp.int32)
    typ = edge_type.astype(jnp.int16)
    zb = jax.lax.bitcast_convert_type(
        z.astype(jnp.bfloat16).reshape(z.shape[0], HW, 2), jnp.int32)
    rb = jax.lax.bitcast_convert_type(
        rel_emb.astype(jnp.bfloat16).reshape(rel_emb.shape[0], HW, 2),
        jnp.int32)
    return _distmult_sc(src, dst, typ, zb, rb)


# speedup vs baseline: 2.0015x; 1.2090x over previous
import functools

import jax
import jax.numpy as jnp
from jax import lax
from jax.experimental import pallas as pl
from jax.experimental.pallas import tpu as pltpu
from jax.experimental.pallas import tpu_sc as plsc

E = 320000
H = 128
R = 500
NN = 10000
NC = 2
NS = 16
NW = NC * NS
EPW = E // NW
K = 80
NCHUNK = EPW // K
G = K // 16
HW = H // 2

_EU = 4
_ILV = plsc.PackFormat.INTERLEAVED

_mesh = plsc.VectorSubcoreMesh(core_axis_name="c", subcore_axis_name="s")


@functools.partial(
    pl.kernel,
    mesh=_mesh,
    out_type=jax.ShapeDtypeStruct((E,), jnp.float32),
    compiler_params=pltpu.CompilerParams(needs_layout_passes=False,
                                         use_tc_tiling_on_sc=False),
    scratch_types=[
        pltpu.VMEM((EPW,), jnp.int32),
        pltpu.VMEM((EPW,), jnp.int32),
        pltpu.VMEM((EPW + 32,), jnp.int16),
        pltpu.VMEM((EPW,), jnp.float32),
        pltpu.VMEM((R, HW), jnp.int32),
        pltpu.VMEM((K, HW), jnp.int32),
        pltpu.VMEM((K, HW), jnp.int32),
        pltpu.VMEM((K, HW), jnp.int32),
        pltpu.VMEM((K, HW), jnp.int32),
        pltpu.VMEM((K, HW), jnp.int32),
        pltpu.VMEM((K, HW), jnp.int32),
        pltpu.SemaphoreType.DMA,
        pltpu.SemaphoreType.DMA,
        pltpu.SemaphoreType.DMA,
        pltpu.SemaphoreType.DMA,
        pltpu.SemaphoreType.DMA,
        pltpu.SemaphoreType.DMA,
    ],
)
def _distmult_sc(src_hbm, dst_hbm, typ_hbm, z_hbm, rel_hbm, out_hbm,
                 sidx_v, didx_v, tidx_v, out_v, rl_all,
                 zsA, zdA, zsB, zdB, zsC, zdC,
                 ssA, sdA, ssB, sdB, ssC, sdC):
    wid = lax.axis_index("s") * NC + lax.axis_index("c")
    row16 = lax.iota(jnp.int32, 16)
    bufs = ((zsA, zdA, ssA, sdA), (zsB, zdB, ssB, sdB), (zsC, zdC, ssC, sdC))

    base = wid * EPW
    pltpu.sync_copy(src_hbm.at[pl.ds(base, EPW)], sidx_v)
    pltpu.sync_copy(dst_hbm.at[pl.ds(base, EPW)], didx_v)
    pltpu.sync_copy(typ_hbm.at[pl.ds(base, EPW)], tidx_v.at[pl.ds(0, EPW)])
    pltpu.sync_copy(rel_hbm, rl_all)

    def start(c, buf):
        zs, zd, s_s, s_d = buf
        off = c * K
        pltpu.async_copy(z_hbm.at[sidx_v.at[pl.ds(off, K)]], zs, s_s)
        pltpu.async_copy(z_hbm.at[didx_v.at[pl.ds(off, K)]], zd, s_d)

    def wait(c, buf):
        zs, zd, s_s, s_d = buf
        off = c * K
        pltpu.make_async_copy(z_hbm.at[sidx_v.at[pl.ds(off, K)]], zs, s_s).wait()
        pltpu.make_async_copy(z_hbm.at[didx_v.at[pl.ds(off, K)]], zd, s_d).wait()

    def compute(c, buf):
        zs, zd = buf[0], buf[1]

        def group_body(g, carry):
            gbase = g * 16

            def edge_blk(eb, acc_out):
                tvec = tidx_v[pl.ds(c * K + gbase + eb * _EU, 32)]
                tv0, tv1 = plsc.unpack(tvec, format=_ILV)
                for u in range(_EU):
                    e16 = eb * _EU + u
                    e = gbase + e16
                    tid = (tv0, tv1)[u % 2][u // 2]
                    prods = []
                    for t in range(H // 32):
                        sl = pl.ds(t * 16, 16)
                        a = plsc.bitcast(zs[e, sl], jnp.bfloat16)
                        r = plsc.bitcast(rl_all[tid, sl], jnp.bfloat16)
                        d = plsc.bitcast(zd[e, sl], jnp.bfloat16)
                        p0, p1 = plsc.unpack(a * r * d, format=_ILV)
                        prods.append(p0)
                        prods.append(p1)
                    while len(prods) > 1:
                        prods = [prods[k] + prods[k + 1]
                                 for k in range(0, len(prods), 2)]
                    s = jnp.sum(prods[0])
                    acc_out = jnp.where(row16 == e16, s, acc_out)
                return acc_out

            acc_out = lax.fori_loop(0, 16 // _EU, edge_blk,
                                    jnp.zeros((16,), jnp.float32))
            out_v[pl.ds(c * K + gbase, 16)] = acc_out
            return carry

        lax.fori_loop(0, G, group_body, 0)

    start(0, bufs[0])
    start(1, bufs[1])

    def chunk_trip(i, carry):
        for par in range(3):
            c = 3 * i + par
            start(c + 2, bufs[(par + 2) % 3])
            wait(c, bufs[par])
            compute(c, bufs[par])
        return carry

    lax.fori_loop(0, (NCHUNK - 2) // 3, chunk_trip, 0)
    for cl in (NCHUNK - 2, NCHUNK - 1):
        wait(cl, bufs[cl % 3])
        compute(cl, bufs[cl % 3])

    pltpu.sync_copy(out_v, out_hbm.at[pl.ds(base, EPW)])


def kernel(z, edge_index, edge_type, rel_emb):
    src = edge_index[0].astype(jnp.int32)
    dst = edge_index[1].astype(jnp.int32)
    typ = edge_type.astype(jnp.int16)
    zb = jax.lax.bitcast_convert_type(
        z.astype(jnp.bfloat16).reshape(z.shape[0], HW, 2), jnp.int32)
    rb = jax.lax.bitcast_convert_type(
        rel_emb.astype(jnp.bfloat16).reshape(rel_emb.shape[0], HW, 2),
        jnp.int32)
    return _distmult_sc(src, dst, typ, zb, rb)
